# Initial kernel scaffold; baseline (speedup 1.0000x reference)
import jax
import jax.numpy as jnp
from jax.experimental import pallas as pl


def kernel(image, su_s, sv_s, c_s, w1_s, b1_s, w2_s, b2_s, w3_s, b3_s, fc_s, su_i, sv_i, c_i, w1_i, b1_i, w2_i, b2_i, w3_i, b3_i, fc_i):
    raise NotImplementedError("write your pallas kernel here")


# speedup vs baseline: 1.1671x; 1.1671x over previous
import functools

import jax
import jax.numpy as jnp
from jax.experimental import pallas as pl
from jax.experimental.pallas import tpu as pltpu

_H = 61
_EPS = 1e-6
_P = 22500
_PBLK = 2048
_NP = 11
_PPAD = _PBLK * _NP


def _phase_mat(n):
    ne = (n + 1) // 2
    rr = jax.lax.broadcasted_iota(jnp.int32, (n, n), 0)
    cc = jax.lax.broadcasted_iota(jnp.int32, (n, n), 1)
    tgt = jnp.where(cc < ne, 2 * cc, 2 * (cc - ne) + 1)
    return jnp.where(rr == tgt, 1.0, 0.0).astype(jnp.float32)


def _ph_slice(d, n, nout):
    ne = (n + 1) // 2
    s = d // 2 if d % 2 == 0 else ne + (d - 1) // 2
    return slice(s, s + nout)


def _bf(x):
    return x.astype(jnp.bfloat16)


def _hist_body(x_ref, prm_ref, bins_ref, mt_ref, o_ref, lg_ref, ku_ref,
               wkv_ref, *, apply_map):
    x = x_ref[0]
    prm = prm_ref[...]

    if apply_map:
        y = jax.lax.dot_general(_bf(mt_ref[0]), _bf(x),
                                (((1,), (0,)), ((), ())),
                                preferred_element_type=jnp.float32)
    else:
        y = x

    y0 = jnp.clip(y[0:1, :], 0.0, 1.0)
    y1 = jnp.clip(y[1:2, :], 0.0, 1.0)
    y2 = jnp.clip(y[2:3, :], 0.0, 1.0)
    lg_ref[0:1, :] = jnp.log(y0 + _EPS)
    lg_ref[1:2, :] = jnp.log(y1 + _EPS)
    lg_ref[2:3, :] = jnp.log(y2 + _EPS)
    lg_ref[3:4, :] = jnp.sqrt(y0 * y0 + y1 * y1 + y2 * y2)

    bins = jnp.broadcast_to(bins_ref[:, 0:1], (_H, _PBLK))
    for c in range(3):
        isu = prm[0, c]
        isv = prm[0, 3 + c]
        cw = prm[0, 6 + c]
        ua, ub = ((0, 1), (1, 0), (2, 0))[c]
        va, vb = ((0, 2), (1, 2), (2, 1))[c]
        for k in range(_NP):
            sl = slice(k * _PBLK, (k + 1) * _PBLK)
            uc = lg_ref[ua:ua + 1, sl] - lg_ref[ub:ub + 1, sl]
            vc = lg_ref[va:va + 1, sl] - lg_ref[vb:vb + 1, sl]
            iy = lg_ref[3:4, sl]
            du = uc - bins
            ku = jnp.exp(-(du * du) * isu)
            dv = vc - bins
            kv = jnp.exp(-(dv * dv) * isv)
            ku_ref[:, sl] = _bf(ku)
            wkv_ref[:, sl] = _bf(kv * (iy * cw))
        hc = jax.lax.dot_general(ku_ref[...], wkv_ref[...],
                                 (((1,), (1,)), ((), ())),
                                 preferred_element_type=jnp.float32)
        o_ref[0, c] = hc


def _hist_call(x3, prm, bins, mt, *, apply_map):
    b = x3.shape[0]
    body = functools.partial(_hist_body, apply_map=apply_map)
    return pl.pallas_call(
        body,
        out_shape=jax.ShapeDtypeStruct((b, 3, _H, _H), jnp.float32),
        grid=(b,),
        in_specs=[
            pl.BlockSpec((1, 3, _PPAD), lambda i: (i, 0, 0)),
            pl.BlockSpec((1, 16), lambda i: (0, 0)),
            pl.BlockSpec((_H, 128), lambda i: (0, 0)),
            pl.BlockSpec((1, 3, 3), lambda i: (i, 0, 0)),
        ],
        out_specs=pl.BlockSpec((1, 3, _H, _H), lambda i: (i, 0, 0, 0)),
        scratch_shapes=[
            pltpu.VMEM((4, _PPAD), jnp.float32),
            pltpu.VMEM((_H, _PPAD), jnp.bfloat16),
            pltpu.VMEM((_H, _PPAD), jnp.bfloat16),
        ],
        compiler_params=pltpu.CompilerParams(
            dimension_semantics=("parallel",),
            vmem_limit_bytes=100 * 1024 * 1024,
        ),
        name="uv_hist",
    )(x3, prm, bins, mt)


def _head_body(h_ref, w1_ref, w2_ref, w3_ref, bia_ref, fc_ref,
               o_ref, s1_ref, p2_ref, s2_ref, p3_ref, *, nout):
    x = h_ref[0]
    g1 = _phase_mat(61)
    g1t = g1.T

    mm = lambda u, v: jnp.dot(u, v, precision=jax.lax.Precision.HIGHEST,
                              preferred_element_type=jnp.float32)
    xpp = [mm(g1t, mm(x[c], g1)) for c in range(3)]
    pieces = []
    for dy in range(5):
        rs = _ph_slice(dy, 61, 29)
        for dx in range(5):
            cs = _ph_slice(dx, 61, 29)
            for c in range(3):
                pieces.append(xpp[c][rs, cs])
    p1 = jnp.stack(pieces, axis=0)
    y1 = jnp.einsum("kpm,ok->pmo", _bf(p1), w1_ref[...],
                    preferred_element_type=jnp.float32)
    y1 = jnp.maximum(y1 + bia_ref[0:1, :128].reshape(1, 1, 128), 0.0)
    s1_ref[...] = y1.reshape(841, 128)

    for dy in range(3):
        for dx in range(3):
            j = 3 * dy + dx
            for oy in range(14):
                base = (2 * oy + dy) * 29 + dx
                p2_ref[14 * oy:14 * (oy + 1), 128 * j:128 * (j + 1)] = (
                    s1_ref[pl.ds(base, 14, 2), :])
    y2 = jnp.dot(_bf(p2_ref[...]), w2_ref[...],
                 preferred_element_type=jnp.float32)
    y2 = jnp.maximum(y2 + bia_ref[1:2, :256], 0.0)
    s2_ref[...] = y2

    for dy in range(2):
        for dx in range(2):
            j = 2 * dy + dx
            for oy in range(13):
                base = (oy + dy) * 14 + dx
                p3_ref[13 * oy:13 * (oy + 1), 256 * j:256 * (j + 1)] = (
                    s2_ref[pl.ds(base, 13, 1), :])
    y3 = jnp.dot(_bf(p3_ref[...]), w3_ref[...],
                 preferred_element_type=jnp.float32)
    y3m = jnp.maximum(y3 + bia_ref[2:3, :], 0.0)

    y3b = _bf(y3m).astype(jnp.float32)
    outs = []
    for i in range(nout):
        outs.append(jnp.abs(jnp.sum(fc_ref[i].astype(jnp.float32) * y3b)))

    lane = jax.lax.broadcasted_iota(jnp.int32, (1, 1, nout), 2)
    row = jnp.zeros((1, 1, nout), jnp.float32)
    for i in range(nout):
        row = jnp.where(lane == i, outs[i], row)
    o_ref[...] = row


def _head_call(hist, w1a, w2m, w3m, bia, fcr, *, nout):
    b = hist.shape[0]
    body = functools.partial(_head_body, nout=nout)
    return pl.pallas_call(
        body,
        out_shape=jax.ShapeDtypeStruct((b, 1, nout), jnp.float32),
        grid=(b,),
        in_specs=[
            pl.BlockSpec((1, 3, _H, _H), lambda i: (i, 0, 0, 0)),
            pl.BlockSpec((128, 75), lambda i: (0, 0)),
            pl.BlockSpec((1152, 256), lambda i: (0, 0)),
            pl.BlockSpec((1024, 512), lambda i: (0, 0)),
            pl.BlockSpec((3, 512), lambda i: (0, 0)),
            pl.BlockSpec((nout, 169, 512), lambda i: (0, 0, 0)),
        ],
        out_specs=pl.BlockSpec((1, 1, nout), lambda i: (i, 0, 0)),
        scratch_shapes=[
            pltpu.VMEM((841, 128), jnp.float32),
            pltpu.VMEM((196, 1152), jnp.float32),
            pltpu.VMEM((196, 256), jnp.float32),
            pltpu.VMEM((169, 1024), jnp.float32),
        ],
        compiler_params=pltpu.CompilerParams(
            dimension_semantics=("parallel",),
            vmem_limit_bytes=110 * 1024 * 1024,
        ),
        name="cnn_head",
    )(hist, w1a, w2m, w3m, bia, fcr)


def _prep_head(w1, b1, w2, b2, w3, b3, fc, nout):
    w1a = _bf(w1.transpose(0, 2, 3, 1).reshape(128, 75))
    w2m = _bf(w2.transpose(2, 3, 1, 0).reshape(1152, 256))
    w3m = _bf(w3.transpose(2, 3, 1, 0).reshape(1024, 512))
    bia = jnp.stack([
        jnp.pad(b1, (0, 512 - 128)),
        jnp.pad(b2, (0, 512 - 256)),
        b3,
    ], axis=0)
    fcr = _bf(fc.reshape(nout, 512, 169).transpose(0, 2, 1))
    return w1a, w2m, w3m, bia, fcr


def _prep_prm(su, sv, c):
    return jnp.concatenate(
        [1.0 / (su * su), 1.0 / (sv * sv), c, jnp.zeros((7,), jnp.float32)]
    ).reshape(1, 16)


def kernel(image, su_s, sv_s, c_s, w1_s, b1_s, w2_s, b2_s, w3_s, b3_s, fc_s,
           su_i, sv_i, c_i, w1_i, b1_i, w2_i, b2_i, w3_i, b3_i, fc_i):
    b = image.shape[0]
    x3 = image.reshape(b, 3, _P)
    x3 = jnp.pad(x3, ((0, 0), (0, 0), (0, _PPAD - _P)))
    bins = jnp.broadcast_to(
        jnp.linspace(-3.0, 3.0, _H).astype(jnp.float32)[:, None], (_H, 128))

    mt_dummy = jnp.zeros((b, 3, 3), jnp.float32)
    hs = _hist_call(x3, _prep_prm(su_s, sv_s, c_s), bins, mt_dummy,
                    apply_map=False)
    hs = hs / (jnp.sum(hs, axis=(1, 2, 3), keepdims=True) + _EPS)
    m9 = _head_call(hs, *_prep_head(w1_s, b1_s, w2_s, b2_s, w3_s, b3_s,
                                    fc_s, 9), nout=9)

    m = jnp.swapaxes(m9.reshape(b, 3, 3), -1, -2)
    n = jnp.max(jnp.sum(jnp.abs(m), axis=-1), axis=-1) + 1e-4
    mn = m / n[:, None, None]

    hi = _hist_call(x3, _prep_prm(su_i, sv_i, c_i), bins,
                    mn, apply_map=True)
    hi = hi / (jnp.sum(hi, axis=(1, 2, 3), keepdims=True) + _EPS)
    ill = _head_call(hi, *_prep_head(w1_i, b1_i, w2_i, b2_i, w3_i, b3_i,
                                     fc_i, 3), nout=3)
    return jnp.einsum('bij,bj->bi', jnp.linalg.inv(mn), ill.reshape(b, 3))


# speedup vs baseline: 1.2333x; 1.0567x over previous
import functools

import jax
import jax.numpy as jnp
from jax.experimental import pallas as pl
from jax.experimental.pallas import tpu as pltpu

_H = 61
_EPS = 1e-6
_P = 22500
_PBLK = 2048
_NP = 11
_PPAD = _PBLK * _NP


def _phase_mat(n):
    ne = (n + 1) // 2
    rr = jax.lax.broadcasted_iota(jnp.int32, (n, n), 0)
    cc = jax.lax.broadcasted_iota(jnp.int32, (n, n), 1)
    tgt = jnp.where(cc < ne, 2 * cc, 2 * (cc - ne) + 1)
    return jnp.where(rr == tgt, 1.0, 0.0).astype(jnp.float32)


def _ph_slice(d, n, nout):
    ne = (n + 1) // 2
    s = d // 2 if d % 2 == 0 else ne + (d - 1) // 2
    return slice(s, s + nout)


def _bf(x):
    return x.astype(jnp.bfloat16)


def _hist_body(x_ref, prm_ref, bins_ref, mt_ref, o_ref, lg_ref, ku_ref,
               wkv_ref, *, apply_map):
    x = x_ref[0]
    prm = prm_ref[...]

    if apply_map:
        y = jax.lax.dot_general(_bf(mt_ref[0]), _bf(x),
                                (((1,), (0,)), ((), ())),
                                preferred_element_type=jnp.float32)
    else:
        y = x

    y0 = jnp.clip(y[0:1, :], 0.0, 1.0)
    y1 = jnp.clip(y[1:2, :], 0.0, 1.0)
    y2 = jnp.clip(y[2:3, :], 0.0, 1.0)
    lg_ref[0:1, :] = jnp.log(y0 + _EPS)
    lg_ref[1:2, :] = jnp.log(y1 + _EPS)
    lg_ref[2:3, :] = jnp.log(y2 + _EPS)
    lg_ref[3:4, :] = jnp.sqrt(y0 * y0 + y1 * y1 + y2 * y2)

    bins = jnp.broadcast_to(bins_ref[:, 0:1], (_H, _PBLK))
    for c in range(3):
        nisu = prm[0, 9 + c]
        nisv = prm[0, 12 + c]
        cw = prm[0, 6 + c]
        ua, ub = ((0, 1), (1, 0), (2, 0))[c]
        va, vb = ((0, 2), (1, 2), (2, 1))[c]
        for k in range(_NP):
            sl = slice(k * _PBLK, (k + 1) * _PBLK)
            uc = lg_ref[ua:ua + 1, sl] - lg_ref[ub:ub + 1, sl]
            vc = lg_ref[va:va + 1, sl] - lg_ref[vb:vb + 1, sl]
            iy = lg_ref[3:4, sl]
            du = uc - bins
            ku = jnp.exp((du * du) * nisu)
            dv = vc - bins
            kv = jnp.exp((dv * dv) * nisv)
            ku_ref[:, sl] = _bf(ku)
            wkv_ref[:, sl] = _bf(kv * (iy * cw))
        hc = jax.lax.dot_general(ku_ref[...], wkv_ref[...],
                                 (((1,), (1,)), ((), ())),
                                 preferred_element_type=jnp.float32)
        o_ref[0, c] = hc


def _hist_call(x3, prm, bins, mt, *, apply_map):
    b = x3.shape[0]
    body = functools.partial(_hist_body, apply_map=apply_map)
    return pl.pallas_call(
        body,
        out_shape=jax.ShapeDtypeStruct((b, 3, _H, _H), jnp.float32),
        grid=(b,),
        in_specs=[
            pl.BlockSpec((1, 3, _PPAD), lambda i: (i, 0, 0)),
            pl.BlockSpec((1, 16), lambda i: (0, 0)),
            pl.BlockSpec((_H, 128), lambda i: (0, 0)),
            pl.BlockSpec((1, 3, 3), lambda i: (i, 0, 0)),
        ],
        out_specs=pl.BlockSpec((1, 3, _H, _H), lambda i: (i, 0, 0, 0)),
        scratch_shapes=[
            pltpu.VMEM((4, _PPAD), jnp.float32),
            pltpu.VMEM((_H, _PPAD), jnp.bfloat16),
            pltpu.VMEM((_H, _PPAD), jnp.bfloat16),
        ],
        compiler_params=pltpu.CompilerParams(
            dimension_semantics=("parallel",),
            vmem_limit_bytes=100 * 1024 * 1024,
        ),
        name="uv_hist",
    )(x3, prm, bins, mt)


def _head_body(h_ref, w1_ref, w2_ref, w3_ref, bia_ref, fc_ref,
               o_ref, s1_ref, p2_ref, s2_ref, p3_ref, *, nout):
    x = h_ref[0]
    g1 = _phase_mat(61)
    g1t = g1.T

    mm = lambda u, v: jnp.dot(u, v, precision=jax.lax.Precision.HIGHEST,
                              preferred_element_type=jnp.float32)
    xpp = [mm(g1t, mm(x[c], g1)) for c in range(3)]
    pieces = []
    for dy in range(5):
        rs = _ph_slice(dy, 61, 29)
        for dx in range(5):
            cs = _ph_slice(dx, 61, 29)
            for c in range(3):
                pieces.append(xpp[c][rs, cs])
    p1 = jnp.stack(pieces, axis=0)
    y1 = jnp.einsum("kpm,ok->pmo", _bf(p1), w1_ref[...],
                    preferred_element_type=jnp.float32)
    y1 = jnp.maximum(y1 + bia_ref[0:1, :128].reshape(1, 1, 128), 0.0)
    s1_ref[...] = y1.reshape(841, 128)

    for dy in range(3):
        for dx in range(3):
            j = 3 * dy + dx
            for oy in range(14):
                base = (2 * oy + dy) * 29 + dx
                p2_ref[14 * oy:14 * (oy + 1), 128 * j:128 * (j + 1)] = (
                    s1_ref[pl.ds(base, 14, 2), :])
    y2 = jnp.dot(_bf(p2_ref[...]), w2_ref[...],
                 preferred_element_type=jnp.float32)
    y2 = jnp.maximum(y2 + bia_ref[1:2, :256], 0.0)
    s2_ref[...] = y2

    for dy in range(2):
        for dx in range(2):
            j = 2 * dy + dx
            for oy in range(13):
                base = (oy + dy) * 14 + dx
                p3_ref[13 * oy:13 * (oy + 1), 256 * j:256 * (j + 1)] = (
                    s2_ref[pl.ds(base, 13, 1), :])
    y3 = jnp.dot(_bf(p3_ref[...]), w3_ref[...],
                 preferred_element_type=jnp.float32)
    y3m = jnp.maximum(y3 + bia_ref[2:3, :], 0.0)

    y3b = _bf(y3m).astype(jnp.float32)
    outs = []
    for i in range(nout):
        outs.append(jnp.abs(jnp.sum(fc_ref[i].astype(jnp.float32) * y3b)))

    lane = jax.lax.broadcasted_iota(jnp.int32, (1, 1, nout), 2)
    row = jnp.zeros((1, 1, nout), jnp.float32)
    for i in range(nout):
        row = jnp.where(lane == i, outs[i], row)
    o_ref[...] = row


def _head_call(hist, w1a, w2m, w3m, bia, fcr, *, nout):
    b = hist.shape[0]
    body = functools.partial(_head_body, nout=nout)
    return pl.pallas_call(
        body,
        out_shape=jax.ShapeDtypeStruct((b, 1, nout), jnp.float32),
        grid=(b,),
        in_specs=[
            pl.BlockSpec((1, 3, _H, _H), lambda i: (i, 0, 0, 0)),
            pl.BlockSpec((128, 75), lambda i: (0, 0)),
            pl.BlockSpec((1152, 256), lambda i: (0, 0)),
            pl.BlockSpec((1024, 512), lambda i: (0, 0)),
            pl.BlockSpec((3, 512), lambda i: (0, 0)),
            pl.BlockSpec((nout, 169, 512), lambda i: (0, 0, 0)),
        ],
        out_specs=pl.BlockSpec((1, 1, nout), lambda i: (i, 0, 0)),
        scratch_shapes=[
            pltpu.VMEM((841, 128), jnp.float32),
            pltpu.VMEM((196, 1152), jnp.float32),
            pltpu.VMEM((196, 256), jnp.float32),
            pltpu.VMEM((169, 1024), jnp.float32),
        ],
        compiler_params=pltpu.CompilerParams(
            dimension_semantics=("parallel",),
            vmem_limit_bytes=110 * 1024 * 1024,
        ),
        name="cnn_head",
    )(hist, w1a, w2m, w3m, bia, fcr)


def _prep_head(w1, b1, w2, b2, w3, b3, fc, nout):
    w1a = _bf(w1.transpose(0, 2, 3, 1).reshape(128, 75))
    w2m = _bf(w2.transpose(2, 3, 1, 0).reshape(1152, 256))
    w3m = _bf(w3.transpose(2, 3, 1, 0).reshape(1024, 512))
    bia = jnp.stack([
        jnp.pad(b1, (0, 512 - 128)),
        jnp.pad(b2, (0, 512 - 256)),
        b3,
    ], axis=0)
    fcr = _bf(fc.reshape(nout, 512, 169).transpose(0, 2, 1))
    return w1a, w2m, w3m, bia, fcr


def _prep_prm(su, sv, c):
    isu = 1.0 / (su * su)
    isv = 1.0 / (sv * sv)
    return jnp.concatenate(
        [isu, isv, c, -isu, -isv, jnp.zeros((1,), jnp.float32)]
    ).reshape(1, 16)


def kernel(image, su_s, sv_s, c_s, w1_s, b1_s, w2_s, b2_s, w3_s, b3_s, fc_s,
           su_i, sv_i, c_i, w1_i, b1_i, w2_i, b2_i, w3_i, b3_i, fc_i):
    b = image.shape[0]
    x3 = image.reshape(b, 3, _P)
    x3 = jnp.pad(x3, ((0, 0), (0, 0), (0, _PPAD - _P)))
    bins = jnp.broadcast_to(
        jnp.linspace(-3.0, 3.0, _H).astype(jnp.float32)[:, None], (_H, 128))

    mt_dummy = jnp.zeros((b, 3, 3), jnp.float32)
    hs = _hist_call(x3, _prep_prm(su_s, sv_s, c_s), bins, mt_dummy,
                    apply_map=False)
    hs = hs / (jnp.sum(hs, axis=(1, 2, 3), keepdims=True) + _EPS)
    m9 = _head_call(hs, *_prep_head(w1_s, b1_s, w2_s, b2_s, w3_s, b3_s,
                                    fc_s, 9), nout=9)

    m = jnp.swapaxes(m9.reshape(b, 3, 3), -1, -2)
    n = jnp.max(jnp.sum(jnp.abs(m), axis=-1), axis=-1) + 1e-4
    mn = m / n[:, None, None]

    hi = _hist_call(x3, _prep_prm(su_i, sv_i, c_i), bins,
                    mn, apply_map=True)
    hi = hi / (jnp.sum(hi, axis=(1, 2, 3), keepdims=True) + _EPS)
    ill = _head_call(hi, *_prep_head(w1_i, b1_i, w2_i, b2_i, w3_i, b3_i,
                                     fc_i, 3), nout=3)
    return jnp.einsum('bij,bj->bi', jnp.linalg.inv(mn), ill.reshape(b, 3))
